# Initial kernel scaffold; baseline (speedup 1.0000x reference)
import jax
import jax.numpy as jnp
from jax.experimental import pallas as pl


def kernel(c, q, ch, qh, word_table, char_table):
    raise NotImplementedError("write your pallas kernel here")


# speedup vs baseline: 3.3948x; 3.3948x over previous
import functools

import jax
import jax.numpy as jnp
from jax import lax
from jax.experimental import pallas as pl
from jax.experimental.pallas import tpu as pltpu
from jax.experimental.pallas import tpu_sc as plsc

B = 1024
C_LEN = 50
Q_LEN = 20
CHAR_LIMIT = 16
CHAR_DIM = 32
CH_ROWS = C_LEN * CHAR_LIMIT
QH_ROWS = Q_LEN * CHAR_LIMIT
ROWS_PER_B = CH_ROWS + QH_ROWS
GCHUNK = 128
PAD_ROWS = 1152
NG = PAD_ROWS // GCHUNK


def _sc_gather(ch_flat, qh_flat, table):
  info = plsc.get_sparse_core_info()
  nc, ns = info.num_cores, info.num_subcores
  nw = nc * ns
  b_per_w = B // nw

  mesh = plsc.VectorSubcoreMesh(core_axis_name="c", subcore_axis_name="s")

  @functools.partial(
      pl.kernel,
      mesh=mesh,
      compiler_params=pltpu.CompilerParams(use_tc_tiling_on_sc=False),
      out_type=jax.ShapeDtypeStruct((B * ROWS_PER_B, CHAR_DIM), jnp.float32),
      scratch_types=[
          pltpu.VMEM((PAD_ROWS,), jnp.int32),
          pltpu.VMEM((PAD_ROWS, CHAR_DIM), jnp.float32),
          pltpu.SemaphoreType.DMA,
      ],
  )
  def k(ch_hbm, qh_hbm, table_hbm, out_hbm, idx_v, rows_v, gsem):
    wid = lax.axis_index("s") * nc + lax.axis_index("c")
    base = wid * b_per_w

    zeros16 = jnp.zeros((16,), jnp.int32)
    idx_v[pl.ds(ROWS_PER_B, 16)] = zeros16
    idx_v[pl.ds(ROWS_PER_B + 16, 16)] = zeros16

    def body(i, carry):
      b = base + i
      pltpu.sync_copy(ch_hbm.at[pl.ds(b * CH_ROWS, CH_ROWS)],
                      idx_v.at[pl.ds(0, CH_ROWS)])
      pltpu.sync_copy(qh_hbm.at[pl.ds(b * QH_ROWS, QH_ROWS)],
                      idx_v.at[pl.ds(CH_ROWS, QH_ROWS)])
      copies = [
          pltpu.async_copy(
              table_hbm.at[idx_v.at[pl.ds(j * GCHUNK, GCHUNK)]],
              rows_v.at[pl.ds(j * GCHUNK, GCHUNK)],
              gsem,
          )
          for j in range(NG)
      ]
      for cp in copies:
        cp.wait()
      pltpu.sync_copy(rows_v.at[pl.ds(0, ROWS_PER_B)],
                      out_hbm.at[pl.ds(b * ROWS_PER_B, ROWS_PER_B)])
      return carry

    lax.fori_loop(0, b_per_w, body, 0)

  return k(ch_flat, qh_flat, table)


def kernel(c, q, ch, qh, word_table, char_table):
  ch_flat = ch.reshape(-1).astype(jnp.int32)
  qh_flat = qh.reshape(-1).astype(jnp.int32)
  out = _sc_gather(ch_flat, qh_flat, char_table)
  return out.reshape(B, C_LEN + Q_LEN, CHAR_LIMIT, CHAR_DIM)


# speedup vs baseline: 4.5385x; 1.3369x over previous
import functools

import jax
import jax.numpy as jnp
from jax import lax
from jax.experimental import pallas as pl
from jax.experimental.pallas import tpu as pltpu
from jax.experimental.pallas import tpu_sc as plsc

B = 1024
C_LEN = 50
Q_LEN = 20
CHAR_LIMIT = 16
CHAR_DIM = 32
CH_ROWS = C_LEN * CHAR_LIMIT
QH_ROWS = Q_LEN * CHAR_LIMIT
ROWS_PER_B = CH_ROWS + QH_ROWS
GCHUNK = 128


def _sc_gather(ch_flat, qh_flat, table):
  info = plsc.get_sparse_core_info()
  nc, ns = info.num_cores, info.num_subcores
  nw = nc * ns
  b_per_w = B // nw
  ch_w = b_per_w * CH_ROWS
  qh_w = b_per_w * QH_ROWS

  mesh = plsc.VectorSubcoreMesh(core_axis_name="c", subcore_axis_name="s")

  @functools.partial(
      pl.kernel,
      mesh=mesh,
      compiler_params=pltpu.CompilerParams(use_tc_tiling_on_sc=False),
      out_type=jax.ShapeDtypeStruct((B * ROWS_PER_B, CHAR_DIM), jnp.float32),
      scratch_types=[
          pltpu.VMEM((ch_w + qh_w,), jnp.int32),
          pltpu.VMEM((ROWS_PER_B, CHAR_DIM), jnp.float32),
          pltpu.VMEM((ROWS_PER_B, CHAR_DIM), jnp.float32),
          pltpu.SemaphoreType.DMA,
          pltpu.SemaphoreType.DMA,
          pltpu.SemaphoreType.DMA,
          pltpu.SemaphoreType.DMA,
      ],
  )
  def k(ch_hbm, qh_hbm, table_hbm, out_hbm, idx_v, rows0, rows1,
        gsem0, gsem1, wsem0, wsem1):
    wid = lax.axis_index("s") * nc + lax.axis_index("c")
    base = wid * b_per_w

    pltpu.sync_copy(ch_hbm.at[pl.ds(base * CH_ROWS, ch_w)],
                    idx_v.at[pl.ds(0, ch_w)])
    pltpu.sync_copy(qh_hbm.at[pl.ds(base * QH_ROWS, qh_w)],
                    idx_v.at[pl.ds(ch_w, qh_w)])

    def fire_gathers(i, rows, gsem):
      ch_base = i * CH_ROWS
      qh_base = ch_w + i * QH_ROWS
      for j in range(6):
        pltpu.async_copy(
            table_hbm.at[idx_v.at[pl.ds(ch_base + j * GCHUNK, GCHUNK)]],
            rows.at[pl.ds(j * GCHUNK, GCHUNK)], gsem)
      pltpu.async_copy(
          table_hbm.at[idx_v.at[pl.ds(ch_base + 6 * GCHUNK, CH_ROWS - 6 * GCHUNK)]],
          rows.at[pl.ds(6 * GCHUNK, CH_ROWS - 6 * GCHUNK)], gsem)
      for j in range(2):
        pltpu.async_copy(
            table_hbm.at[idx_v.at[pl.ds(qh_base + j * GCHUNK, GCHUNK)]],
            rows.at[pl.ds(CH_ROWS + j * GCHUNK, GCHUNK)], gsem)
      pltpu.async_copy(
          table_hbm.at[idx_v.at[pl.ds(qh_base + 2 * GCHUNK, QH_ROWS - 2 * GCHUNK)]],
          rows.at[pl.ds(CH_ROWS + 2 * GCHUNK, QH_ROWS - 2 * GCHUNK)], gsem)

    def out_slice(i):
      return out_hbm.at[pl.ds((base + i) * ROWS_PER_B, ROWS_PER_B)]

    fire_gathers(0, rows0, gsem0)

    def phase(i, k_, rows_cur, gsem_cur, wsem_cur, rows_oth, gsem_oth,
              wsem_oth, first, last):
      if not first:
        pltpu.make_async_copy(rows_oth, out_slice(i - 1), wsem_oth).wait()
      else:
        @pl.when(k_ > 0)
        def _():
          pltpu.make_async_copy(rows_oth, out_slice(i - 1), wsem_oth).wait()
      if last:
        @pl.when(k_ < b_per_w // 2 - 1)
        def _():
          fire_gathers(i + 1, rows_oth, gsem_oth)
      else:
        fire_gathers(i + 1, rows_oth, gsem_oth)
      pltpu.make_async_copy(rows_cur, out_slice(i), gsem_cur).wait()
      pltpu.async_copy(rows_cur, out_slice(i), wsem_cur)

    def body(k_, carry):
      phase(2 * k_, k_, rows0, gsem0, wsem0, rows1, gsem1, wsem1,
            first=True, last=False)
      phase(2 * k_ + 1, k_, rows1, gsem1, wsem1, rows0, gsem0, wsem0,
            first=False, last=True)
      return carry

    lax.fori_loop(0, b_per_w // 2, body, 0)
    pltpu.make_async_copy(rows1, out_slice(b_per_w - 1), wsem1).wait()

  return k(ch_flat, qh_flat, table)


def kernel(c, q, ch, qh, word_table, char_table):
  ch_flat = ch.reshape(-1).astype(jnp.int32)
  qh_flat = qh.reshape(-1).astype(jnp.int32)
  out = _sc_gather(ch_flat, qh_flat, char_table)
  return out.reshape(B, C_LEN + Q_LEN, CHAR_LIMIT, CHAR_DIM)
